# baseline (speedup 1.0000x reference)
import jax
import jax.numpy as jnp
from jax import lax
from jax.experimental import pallas as pl
from jax.experimental.pallas import tpu as pltpu, tpu_sc as plsc

DIM = 64
LSEQ = 50
BATCH = 4096
NC, NS = 2, 16
NW = NC * NS
BB = 128
NBT = BATCH // BB
UNITS = LSEQ * NBT
UPW = UNITS // NW
LPAD = 56


def _body(src_tbl, tgt_tbl, src_idxT, tgt_idxT, src_out, tgt_out,
          idx_s, idx_t, rv_s, rv_t, rt_s, rt_t,
          sg_s, sg_t, so_s, so_t, si_s, si_t):
    w = lax.axis_index("s") * NC + lax.axis_index("c")
    iota = lax.iota(jnp.int32, 16)

    streams = (
        (src_idxT, src_tbl, src_out, idx_s, rv_s, rt_s, sg_s, so_s, si_s),
        (tgt_idxT, tgt_tbl, tgt_out, idx_t, rv_t, rt_t, sg_t, so_t, si_t),
    )

    def coords(k):
        g = k * NW + w
        return lax.div(g, NBT), lax.rem(g, NBT) * BB

    for idxT, tbl, out, idxv, rv, rt, sg, so, si in streams:
        l0, b0 = coords(0)
        pltpu.sync_copy(idxT.at[l0, pl.ds(b0, BB)], idxv.at[0])
        pltpu.async_copy(tbl.at[idxv.at[0]], rv.at[0], sg)
        l1, b1 = coords(1)
        pltpu.async_copy(idxT.at[l1, pl.ds(b1, BB)], idxv.at[1], si)

    def body(i, carry):
        p = lax.rem(i, 2)
        q = 1 - p
        for idxT, tbl, out, idxv, rv, rt, sg, so, si in streams:
            l, b0 = coords(i)
            pltpu.make_async_copy(tbl.at[pl.ds(0, BB)], rv.at[p], sg).wait()
            @pl.when(i > 0)
            def _():
                pltpu.make_async_copy(rt.at[q],
                                      out.at[0, :, pl.ds(0, BB)], so).wait()
            pltpu.make_async_copy(idxT.at[0, pl.ds(0, BB)], idxv.at[q], si).wait()
            l2, b2 = coords(i + 2)
            pltpu.async_copy(idxT.at[l2, pl.ds(b2, BB)], idxv.at[p], si)
            @pl.when(i + 1 < UPW)
            def _():
                pltpu.async_copy(tbl.at[idxv.at[q]], rv.at[q], sg)
            def trans_d(d, c2):
                dsplat = jnp.full((16,), d, jnp.int32)
                psplat = jnp.full((16,), p, jnp.int32)
                for bg in range(BB // 16):
                    v = plsc.load_gather(rv, [psplat, bg * 16 + iota, dsplat])
                    rt[p, d, pl.ds(bg * 16, 16)] = v
                return c2
            lax.fori_loop(0, DIM, trans_d, 0)
            pltpu.async_copy(rt.at[p], out.at[l, :, pl.ds(b0, BB)], so)
        return carry

    lax.fori_loop(0, UPW, body, 0)

    lastp = (UPW - 1) % 2
    for idxT, tbl, out, idxv, rv, rt, sg, so, si in streams:
        pltpu.make_async_copy(rt.at[lastp], out.at[0, :, pl.ds(0, BB)], so).wait()
        pltpu.make_async_copy(idxT.at[0, pl.ds(0, BB)], idxv.at[lastp], si).wait()


@jax.jit
def kernel(src_indices, tgt_indices, src_table, tgt_table):
    b, l = src_indices.shape
    siT = jnp.pad(src_indices.astype(jnp.int32).T, ((0, LPAD - LSEQ), (0, 0)))
    tiT = jnp.pad(tgt_indices.astype(jnp.int32).T, ((0, LPAD - LSEQ), (0, 0)))
    f = pl.kernel(
        _body,
        out_type=(
            jax.ShapeDtypeStruct((LSEQ, DIM, BATCH), jnp.float32),
            jax.ShapeDtypeStruct((LSEQ, DIM, BATCH), jnp.float32),
        ),
        mesh=plsc.VectorSubcoreMesh(core_axis_name="c", subcore_axis_name="s"),
        scratch_types=[
            pltpu.VMEM((2, BB), jnp.int32),
            pltpu.VMEM((2, BB), jnp.int32),
            pltpu.VMEM((2, BB, DIM), jnp.float32),
            pltpu.VMEM((2, BB, DIM), jnp.float32),
            pltpu.VMEM((2, DIM, BB), jnp.float32),
            pltpu.VMEM((2, DIM, BB), jnp.float32),
            pltpu.SemaphoreType.DMA,
            pltpu.SemaphoreType.DMA,
            pltpu.SemaphoreType.DMA,
            pltpu.SemaphoreType.DMA,
            pltpu.SemaphoreType.DMA,
            pltpu.SemaphoreType.DMA,
        ],
        compiler_params=pltpu.CompilerParams(use_tc_tiling_on_sc=False,
                                             needs_layout_passes=False),
    )
    out_s, out_t = f(src_table, tgt_table, siT, tiT)
    return (jnp.transpose(out_s, (2, 0, 1)), jnp.transpose(out_t, (2, 0, 1)))


# speedup vs baseline: 1.4685x; 1.4685x over previous
import jax
import jax.numpy as jnp
from jax import lax
from jax.experimental import pallas as pl
from jax.experimental.pallas import tpu as pltpu, tpu_sc as plsc

DIM = 64
LSEQ = 50
BATCH = 4096
NC, NS = 2, 16
NW = NC * NS
BB = 128
NBT = BATCH // BB
UNITS = LSEQ * NBT
UPW = UNITS // NW
LPAD = 56


def _body(src_tbl, tgt_tbl, src_idxT, tgt_idxT, src_out, tgt_out,
          idx_s, idx_t, rv_s, rv_t, rt_s, rt_t,
          sg_s, sg_t, so_s, so_t, si_s, si_t):
    w = lax.axis_index("s") * NC + lax.axis_index("c")
    iota = lax.iota(jnp.int32, 16)

    streams = (
        (src_idxT, src_tbl, src_out, idx_s, rv_s, rt_s, sg_s, so_s, si_s),
        (tgt_idxT, tgt_tbl, tgt_out, idx_t, rv_t, rt_t, sg_t, so_t, si_t),
    )

    def coords(k):
        g = k * NW + w
        return lax.div(g, NBT), lax.rem(g, NBT) * BB

    for idxT, tbl, out, idxv, rv, rt, sg, so, si in streams:
        l0, b0 = coords(0)
        pltpu.sync_copy(idxT.at[l0, pl.ds(b0, BB)], idxv.at[0])
        pltpu.async_copy(tbl.at[idxv.at[0]], rv.at[0], sg)
        l1, b1 = coords(1)
        pltpu.async_copy(idxT.at[l1, pl.ds(b1, BB)], idxv.at[1], si)

    def body(i, carry):
        p = lax.rem(i, 2)
        q = 1 - p
        for idxT, tbl, out, idxv, rv, rt, sg, so, si in streams:
            l, b0 = coords(i)
            pltpu.make_async_copy(tbl.at[pl.ds(0, BB)], rv.at[p], sg).wait()
            @pl.when(i > 0)
            def _():
                pltpu.make_async_copy(rt.at[q],
                                      out.at[0, :, pl.ds(0, BB)], so).wait()
            pltpu.make_async_copy(idxT.at[0, pl.ds(0, BB)], idxv.at[q], si).wait()
            l2, b2 = coords(i + 2)
            pltpu.async_copy(idxT.at[l2, pl.ds(b2, BB)], idxv.at[p], si)
            @pl.when(i + 1 < UPW)
            def _():
                pltpu.async_copy(tbl.at[idxv.at[q]], rv.at[q], sg)
            psplat = jnp.full((16,), p, jnp.int32)

            @plsc.parallel_loop(0, DIM, unroll=8)
            def _(d):
                dsplat = jnp.full((16,), d, jnp.int32)
                for bg in range(BB // 16):
                    v = plsc.load_gather(rv, [psplat, bg * 16 + iota, dsplat])
                    rt[p, d, pl.ds(bg * 16, 16)] = v
            pltpu.async_copy(rt.at[p], out.at[l, :, pl.ds(b0, BB)], so)
        return carry

    lax.fori_loop(0, UPW, body, 0)

    lastp = (UPW - 1) % 2
    for idxT, tbl, out, idxv, rv, rt, sg, so, si in streams:
        pltpu.make_async_copy(rt.at[lastp], out.at[0, :, pl.ds(0, BB)], so).wait()
        pltpu.make_async_copy(idxT.at[0, pl.ds(0, BB)], idxv.at[lastp], si).wait()


@jax.jit
def kernel(src_indices, tgt_indices, src_table, tgt_table):
    b, l = src_indices.shape
    siT = jnp.pad(src_indices.astype(jnp.int32).T, ((0, LPAD - LSEQ), (0, 0)))
    tiT = jnp.pad(tgt_indices.astype(jnp.int32).T, ((0, LPAD - LSEQ), (0, 0)))
    f = pl.kernel(
        _body,
        out_type=(
            jax.ShapeDtypeStruct((LSEQ, DIM, BATCH), jnp.float32),
            jax.ShapeDtypeStruct((LSEQ, DIM, BATCH), jnp.float32),
        ),
        mesh=plsc.VectorSubcoreMesh(core_axis_name="c", subcore_axis_name="s"),
        scratch_types=[
            pltpu.VMEM((2, BB), jnp.int32),
            pltpu.VMEM((2, BB), jnp.int32),
            pltpu.VMEM((2, BB, DIM), jnp.float32),
            pltpu.VMEM((2, BB, DIM), jnp.float32),
            pltpu.VMEM((2, DIM, BB), jnp.float32),
            pltpu.VMEM((2, DIM, BB), jnp.float32),
            pltpu.SemaphoreType.DMA,
            pltpu.SemaphoreType.DMA,
            pltpu.SemaphoreType.DMA,
            pltpu.SemaphoreType.DMA,
            pltpu.SemaphoreType.DMA,
            pltpu.SemaphoreType.DMA,
        ],
        compiler_params=pltpu.CompilerParams(use_tc_tiling_on_sc=False,
                                             needs_layout_passes=False),
    )
    out_s, out_t = f(src_table, tgt_table, siT, tiT)
    return (jnp.transpose(out_s, (2, 0, 1)), jnp.transpose(out_t, (2, 0, 1)))


# speedup vs baseline: 2.7211x; 1.8530x over previous
import jax
import jax.numpy as jnp
from jax import lax
from jax.experimental import pallas as pl
from jax.experimental.pallas import tpu as pltpu, tpu_sc as plsc

DIM = 64
LSEQ = 50
BATCH = 4096
NC, NS = 2, 16
NW = NC * NS
BB = 128
NBT = BATCH // BB
UNITS = LSEQ * NBT
UPW = UNITS // NW
LPAD = 56


def _body(src_tbl, tgt_tbl, src_idxT, tgt_idxT, src_out, tgt_out,
          idx_s, idx_t, rv_s, rv_t, rt_s, rt_t,
          sg_s, sg_t, so_s, so_t, si_s, si_t):
    w = lax.axis_index("s") * NC + lax.axis_index("c")
    iota = lax.iota(jnp.int32, 16)

    streams = (
        (src_idxT, src_tbl, src_out, idx_s, rv_s, rt_s, sg_s, so_s, si_s),
        (tgt_idxT, tgt_tbl, tgt_out, idx_t, rv_t, rt_t, sg_t, so_t, si_t),
    )

    def coords(k):
        g = k * NW + w
        return lax.div(g, NBT), lax.rem(g, NBT) * BB

    for idxT, tbl, out, idxv, rv, rt, sg, so, si in streams:
        l0, b0 = coords(0)
        pltpu.sync_copy(idxT.at[l0, pl.ds(b0, BB)], idxv.at[0])
        pltpu.async_copy(tbl.at[idxv.at[0]], rv.at[0], sg)
        l1, b1 = coords(1)
        pltpu.async_copy(idxT.at[l1, pl.ds(b1, BB)], idxv.at[1], si)

    def body(i, carry):
        p = lax.rem(i, 2)
        q = 1 - p
        for idxT, tbl, out, idxv, rv, rt, sg, so, si in streams:
            l, b0 = coords(i)
            pltpu.make_async_copy(tbl.at[pl.ds(0, BB)], rv.at[p], sg).wait()
            @pl.when(i > 0)
            def _():
                pltpu.make_async_copy(rt.at[q, :, pl.ds(0, BB)],
                                      out.at[0, :, pl.ds(0, BB)], so).wait()
            pltpu.make_async_copy(idxT.at[0, pl.ds(0, BB)], idxv.at[q], si).wait()
            l2, b2 = coords(i + 2)
            pltpu.async_copy(idxT.at[l2, pl.ds(b2, BB)], idxv.at[p], si)
            @pl.when(i + 1 < UPW)
            def _():
                pltpu.async_copy(tbl.at[idxv.at[q]], rv.at[q], sg)
            psplat = jnp.full((16,), p, jnp.int32)

            @plsc.parallel_loop(0, BB, unroll=8)
            def _(bb):
                bsplat = jnp.full((16,), bb, jnp.int32)
                for dg in range(DIM // 16):
                    v = rv[p, bb, pl.ds(dg * 16, 16)]
                    plsc.store_scatter(rt, [psplat, dg * 16 + iota, bsplat], v)
            pltpu.async_copy(rt.at[p, :, pl.ds(0, BB)],
                             out.at[l, :, pl.ds(b0, BB)], so)
        return carry

    lax.fori_loop(0, UPW, body, 0)

    lastp = (UPW - 1) % 2
    for idxT, tbl, out, idxv, rv, rt, sg, so, si in streams:
        pltpu.make_async_copy(rt.at[lastp, :, pl.ds(0, BB)],
                              out.at[0, :, pl.ds(0, BB)], so).wait()
        pltpu.make_async_copy(idxT.at[0, pl.ds(0, BB)], idxv.at[lastp], si).wait()


@jax.jit
def kernel(src_indices, tgt_indices, src_table, tgt_table):
    b, l = src_indices.shape
    siT = jnp.pad(src_indices.astype(jnp.int32).T, ((0, LPAD - LSEQ), (0, 0)))
    tiT = jnp.pad(tgt_indices.astype(jnp.int32).T, ((0, LPAD - LSEQ), (0, 0)))
    f = pl.kernel(
        _body,
        out_type=(
            jax.ShapeDtypeStruct((LSEQ, DIM, BATCH), jnp.float32),
            jax.ShapeDtypeStruct((LSEQ, DIM, BATCH), jnp.float32),
        ),
        mesh=plsc.VectorSubcoreMesh(core_axis_name="c", subcore_axis_name="s"),
        scratch_types=[
            pltpu.VMEM((2, BB), jnp.int32),
            pltpu.VMEM((2, BB), jnp.int32),
            pltpu.VMEM((2, BB, DIM), jnp.float32),
            pltpu.VMEM((2, BB, DIM), jnp.float32),
            pltpu.VMEM((2, DIM, BB + 1), jnp.float32),
            pltpu.VMEM((2, DIM, BB + 1), jnp.float32),
            pltpu.SemaphoreType.DMA,
            pltpu.SemaphoreType.DMA,
            pltpu.SemaphoreType.DMA,
            pltpu.SemaphoreType.DMA,
            pltpu.SemaphoreType.DMA,
            pltpu.SemaphoreType.DMA,
        ],
        compiler_params=pltpu.CompilerParams(use_tc_tiling_on_sc=False,
                                             needs_layout_passes=False),
    )
    out_s, out_t = f(src_table, tgt_table, siT, tiT)
    return (jnp.transpose(out_s, (2, 0, 1)), jnp.transpose(out_t, (2, 0, 1)))


# speedup vs baseline: 4.0917x; 1.5037x over previous
import jax
import jax.numpy as jnp
from jax import lax
from jax.experimental import pallas as pl
from jax.experimental.pallas import tpu as pltpu, tpu_sc as plsc

DIM = 64
LSEQ = 50
BATCH = 4096
NC, NS = 2, 16
NW = NC * NS
BB = 128
NBT = BATCH // BB
UNITS = LSEQ * NBT
UPW = UNITS // NW
LPAD = 56


def _body(src_tbl, tgt_tbl, src_idxT, tgt_idxT, src_out, tgt_out,
          idx_s, idx_t, rv_s, rv_t, rt_s, rt_t,
          sg_s, sg_t, so_s, so_t, si_s, si_t):
    w = lax.axis_index("s") * NC + lax.axis_index("c")
    iota = lax.iota(jnp.int32, 16)
    dvecs = [k * 16 + iota for k in range(DIM // 16)]
    dgvs = [lax.div(v, 8) for v in dvecs]
    dsvs = [lax.rem(v, 8) for v in dvecs]

    streams = (
        (src_idxT, src_tbl, src_out, idx_s, rv_s, rt_s, sg_s, so_s, si_s),
        (tgt_idxT, tgt_tbl, tgt_out, idx_t, rv_t, rt_t, sg_t, so_t, si_t),
    )

    def coords(k):
        g = k * NW + w
        return lax.div(g, NBT), lax.rem(g, NBT)

    for idxT, tbl, out, idxv, rv, rt, sg, so, si in streams:
        l0, bt0 = coords(0)
        pltpu.sync_copy(idxT.at[l0, bt0], idxv.at[0])
        pltpu.async_copy(tbl.at[idxv.at[0]], rv.at[0], sg)
        l1, bt1 = coords(1)
        pltpu.async_copy(idxT.at[l1, bt1], idxv.at[1], si)

    def body(i, carry):
        p = lax.rem(i, 2)
        q = 1 - p
        for idxT, tbl, out, idxv, rv, rt, sg, so, si in streams:
            l, bt = coords(i)
            pltpu.make_async_copy(tbl.at[pl.ds(0, BB)], rv.at[p], sg).wait()
            @pl.when(i > 0)
            def _():
                pltpu.make_async_copy(rt.at[q, :, :, pl.ds(0, BB)],
                                      out.at[0, :, 0, :, :], so).wait()
            pltpu.make_async_copy(idxT.at[0, 0], idxv.at[q], si).wait()
            l2, bt2 = coords(i + 2)
            pltpu.async_copy(idxT.at[l2, bt2], idxv.at[p], si)
            @pl.when(i + 1 < UPW)
            def _():
                pltpu.async_copy(tbl.at[idxv.at[q]], rv.at[q], sg)
            psplat = jnp.full((16,), p, jnp.int32)

            @plsc.parallel_loop(0, BB, unroll=8)
            def _(bb):
                bsplat = jnp.full((16,), bb, jnp.int32)
                for k in range(DIM // 16):
                    v = rv[p, bb, pl.ds(k * 16, 16)]
                    plsc.store_scatter(rt, [psplat, dgvs[k], dsvs[k], bsplat], v)
            pltpu.async_copy(rt.at[p, :, :, pl.ds(0, BB)],
                             out.at[l, :, bt, :, :], so)
        return carry

    lax.fori_loop(0, UPW, body, 0)

    lastp = (UPW - 1) % 2
    for idxT, tbl, out, idxv, rv, rt, sg, so, si in streams:
        pltpu.make_async_copy(rt.at[lastp, :, :, pl.ds(0, BB)],
                              out.at[0, :, 0, :, :], so).wait()
        pltpu.make_async_copy(idxT.at[0, 0], idxv.at[lastp], si).wait()


@jax.jit
def kernel(src_indices, tgt_indices, src_table, tgt_table):
    siT = jnp.pad(src_indices.astype(jnp.int32).T, ((0, LPAD - LSEQ), (0, 0)))
    tiT = jnp.pad(tgt_indices.astype(jnp.int32).T, ((0, LPAD - LSEQ), (0, 0)))
    siT = siT.reshape(LPAD, NBT, BB)
    tiT = tiT.reshape(LPAD, NBT, BB)
    f = pl.kernel(
        _body,
        out_type=(
            jax.ShapeDtypeStruct((LSEQ, DIM // 8, BATCH // 128, 8, 128),
                                 jnp.float32),
            jax.ShapeDtypeStruct((LSEQ, DIM // 8, BATCH // 128, 8, 128),
                                 jnp.float32),
        ),
        mesh=plsc.VectorSubcoreMesh(core_axis_name="c", subcore_axis_name="s"),
        scratch_types=[
            pltpu.VMEM((2, BB), jnp.int32),
            pltpu.VMEM((2, BB), jnp.int32),
            pltpu.VMEM((2, BB, DIM), jnp.float32),
            pltpu.VMEM((2, BB, DIM), jnp.float32),
            pltpu.VMEM((2, DIM // 8, 8, BB + 1), jnp.float32),
            pltpu.VMEM((2, DIM // 8, 8, BB + 1), jnp.float32),
            pltpu.SemaphoreType.DMA,
            pltpu.SemaphoreType.DMA,
            pltpu.SemaphoreType.DMA,
            pltpu.SemaphoreType.DMA,
            pltpu.SemaphoreType.DMA,
            pltpu.SemaphoreType.DMA,
        ],
        compiler_params=pltpu.CompilerParams(use_tc_tiling_on_sc=False,
                                             needs_layout_passes=False),
    )
    out_s, out_t = f(src_table, tgt_table, siT, tiT)
    out_s = jnp.transpose(out_s, (2, 4, 0, 1, 3)).reshape(BATCH, LSEQ, DIM)
    out_t = jnp.transpose(out_t, (2, 4, 0, 1, 3)).reshape(BATCH, LSEQ, DIM)
    return (out_s, out_t)


# speedup vs baseline: 4.1012x; 1.0023x over previous
import jax
import jax.numpy as jnp
from jax import lax
from jax.experimental import pallas as pl
from jax.experimental.pallas import tpu as pltpu, tpu_sc as plsc

DIM = 64
LSEQ = 50
BATCH = 4096
NC, NS = 2, 16
NW = NC * NS
BB = 128
NBT = BATCH // BB
UNITS = LSEQ * NBT
UPW = UNITS // NW
LPAD = 56


def _body(src_tbl, tgt_tbl, src_idxT, tgt_idxT, src_out, tgt_out,
          idx_s, idx_t, rv_s, rv_t, rt_s, rt_t,
          sg_s, sg_t, so_s, so_t, si_s, si_t):
    w = lax.axis_index("s") * NC + lax.axis_index("c")
    iota = lax.iota(jnp.int32, 16)
    dvecs = [k * 16 + iota for k in range(DIM // 16)]
    dgvs = [lax.div(v, 8) for v in dvecs]
    dsvs = [lax.rem(v, 8) for v in dvecs]

    streams = (
        (src_idxT, src_tbl, src_out, idx_s, rv_s, rt_s, sg_s, so_s, si_s),
        (tgt_idxT, tgt_tbl, tgt_out, idx_t, rv_t, rt_t, sg_t, so_t, si_t),
    )

    def coords(k):
        g = k * NW + w
        return lax.div(g, NBT), lax.rem(g, NBT)

    for idxT, tbl, out, idxv, rv, rt, sg, so, si in streams:
        l0, bt0 = coords(0)
        pltpu.sync_copy(idxT.at[l0, bt0], idxv.at[0])
        pltpu.async_copy(tbl.at[idxv.at[0]], rv.at[0], sg)
        l1, bt1 = coords(1)
        pltpu.async_copy(idxT.at[l1, bt1], idxv.at[1], si)

    def body(i, carry):
        p = lax.rem(i, 2)
        q = 1 - p
        for idxT, tbl, out, idxv, rv, rt, sg, so, si in streams:
            l, bt = coords(i)
            pltpu.make_async_copy(tbl.at[pl.ds(0, BB)], rv.at[p], sg).wait()
            @pl.when(i > 0)
            def _():
                pltpu.make_async_copy(rt.at[q, :, :, pl.ds(0, BB)],
                                      out.at[0, :, 0, :, :], so).wait()
            pltpu.make_async_copy(idxT.at[0, 0], idxv.at[q], si).wait()
            l2, bt2 = coords(i + 2)
            pltpu.async_copy(idxT.at[l2, bt2], idxv.at[p], si)
            @pl.when(i + 1 < UPW)
            def _():
                pltpu.async_copy(tbl.at[idxv.at[q]], rv.at[q], sg)
            psplat = jnp.full((16,), p, jnp.int32)

            @plsc.parallel_loop(0, BB, unroll=16)
            def _(bb):
                bsplat = jnp.full((16,), bb, jnp.int32)
                for k in range(DIM // 16):
                    v = rv[p, bb, pl.ds(k * 16, 16)]
                    plsc.store_scatter(rt, [psplat, dgvs[k], dsvs[k], bsplat], v)
            pltpu.async_copy(rt.at[p, :, :, pl.ds(0, BB)],
                             out.at[l, :, bt, :, :], so)
        return carry

    lax.fori_loop(0, UPW, body, 0)

    lastp = (UPW - 1) % 2
    for idxT, tbl, out, idxv, rv, rt, sg, so, si in streams:
        pltpu.make_async_copy(rt.at[lastp, :, :, pl.ds(0, BB)],
                              out.at[0, :, 0, :, :], so).wait()
        pltpu.make_async_copy(idxT.at[0, 0], idxv.at[lastp], si).wait()


@jax.jit
def kernel(src_indices, tgt_indices, src_table, tgt_table):
    siT = jnp.pad(src_indices.astype(jnp.int32).T, ((0, LPAD - LSEQ), (0, 0)))
    tiT = jnp.pad(tgt_indices.astype(jnp.int32).T, ((0, LPAD - LSEQ), (0, 0)))
    siT = siT.reshape(LPAD, NBT, BB)
    tiT = tiT.reshape(LPAD, NBT, BB)
    f = pl.kernel(
        _body,
        out_type=(
            jax.ShapeDtypeStruct((LSEQ, DIM // 8, BATCH // 128, 8, 128),
                                 jnp.float32),
            jax.ShapeDtypeStruct((LSEQ, DIM // 8, BATCH // 128, 8, 128),
                                 jnp.float32),
        ),
        mesh=plsc.VectorSubcoreMesh(core_axis_name="c", subcore_axis_name="s"),
        scratch_types=[
            pltpu.VMEM((2, BB), jnp.int32),
            pltpu.VMEM((2, BB), jnp.int32),
            pltpu.VMEM((2, BB, DIM), jnp.float32),
            pltpu.VMEM((2, BB, DIM), jnp.float32),
            pltpu.VMEM((2, DIM // 8, 8, BB + 1), jnp.float32),
            pltpu.VMEM((2, DIM // 8, 8, BB + 1), jnp.float32),
            pltpu.SemaphoreType.DMA,
            pltpu.SemaphoreType.DMA,
            pltpu.SemaphoreType.DMA,
            pltpu.SemaphoreType.DMA,
            pltpu.SemaphoreType.DMA,
            pltpu.SemaphoreType.DMA,
        ],
        compiler_params=pltpu.CompilerParams(use_tc_tiling_on_sc=False,
                                             needs_layout_passes=False),
    )
    out_s, out_t = f(src_table, tgt_table, siT, tiT)
    out_s = jnp.transpose(out_s, (2, 4, 0, 1, 3)).reshape(BATCH, LSEQ, DIM)
    out_t = jnp.transpose(out_t, (2, 4, 0, 1, 3)).reshape(BATCH, LSEQ, DIM)
    return (out_s, out_t)
